# Initial kernel scaffold; baseline (speedup 1.0000x reference)
import functools

import jax
import jax.numpy as jnp
from jax import lax
from jax.experimental import pallas as pl
from jax.experimental.pallas import tpu as pltpu
from jax.experimental.pallas import tpu_sc as plsc

N_NODES = 10000
N_EDGES = 320000
D = 128
NG = 128
NL = 3

NC = 2
NS = 16
NW = NC * NS

CHUNK = 128
NCHUNKS = N_EDGES // CHUNK
EDGE_ITERS = -(-NCHUNKS // NW)

ZROWS = 80
NZ = N_NODES // ZROWS
ZITERS = -(-NZ // NS)

ROWS_PER_TILE = N_NODES // NS


def _agg_body(h_hbm, src_hbm, dst_hbm, out_hbm, src_v, dst_v, rows_v, agg_sh, sem):
    c = lax.axis_index("c")
    s = lax.axis_index("s")
    wid = s * NC + c

    def zero_buf(i, carry):
        for j in range(D // 16):
            rows_v[i, pl.ds(j * 16, 16)] = jnp.zeros((16,), jnp.float32)
        return carry

    lax.fori_loop(0, CHUNK, zero_buf, 0)

    def zero_spmem(k, carry):
        cid = s + k * NS

        @pl.when(cid < NZ)
        def _():
            pltpu.sync_copy(rows_v.at[pl.ds(0, ZROWS)],
                            agg_sh.at[pl.ds(cid * ZROWS, ZROWS)])

        return carry

    lax.fori_loop(0, ZITERS, zero_spmem, 0)
    plsc.subcore_barrier()

    def edge_step(k, carry):
        cid = wid + k * NW

        @pl.when(cid < NCHUNKS)
        def _():
            base = cid * CHUNK
            pltpu.sync_copy(src_hbm.at[pl.ds(base, CHUNK)], src_v)
            pltpu.sync_copy(dst_hbm.at[pl.ds(base, CHUNK)], dst_v)
            pltpu.async_copy(h_hbm.at[src_v], rows_v, sem).wait()
            pltpu.sync_copy(rows_v, agg_sh.at[dst_v], add=True)

        return carry

    lax.fori_loop(0, EDGE_ITERS, edge_step, 0)
    plsc.subcore_barrier()

    r0 = s * ROWS_PER_TILE
    pltpu.sync_copy(agg_sh.at[pl.ds(r0, ROWS_PER_TILE)],
                    out_hbm.at[c, pl.ds(r0, ROWS_PER_TILE)])


_agg_call = functools.partial(
    pl.kernel,
    out_type=jax.ShapeDtypeStruct((NC, N_NODES, D), jnp.float32),
    mesh=plsc.VectorSubcoreMesh(core_axis_name="c", subcore_axis_name="s"),
    scratch_types=[
        pltpu.VMEM((CHUNK,), jnp.int32),
        pltpu.VMEM((CHUNK,), jnp.int32),
        pltpu.VMEM((CHUNK, D), jnp.float32),
        pltpu.VMEM_SHARED((N_NODES, D), jnp.float32),
        pltpu.SemaphoreType.DMA,
    ],
)(_agg_body)


BLK = 1000
NBLK = N_NODES // BLK


def _mlp_body(eps_ref, h_ref, p_ref, w1_ref, b1_ref, w2_ref, b2_ref, g_ref,
              hn_ref, pool_ref):
    i = pl.program_id(0)
    h = h_ref[...]
    agg = p_ref[0, :, :] + p_ref[1, :, :]
    z = (1.0 + eps_ref[0, 0]) * h + agg
    z = jnp.maximum(
        jnp.dot(z, w1_ref[...], preferred_element_type=jnp.float32)
        + b1_ref[...], 0.0)
    hn = jnp.maximum(
        jnp.dot(z, w2_ref[...], preferred_element_type=jnp.float32)
        + b2_ref[...], 0.0)
    hn_ref[...] = hn

    cols = lax.broadcasted_iota(jnp.int32, (BLK, NG), 1)
    onehot = jnp.where(g_ref[...] == cols, 1.0, 0.0)
    contrib = lax.dot_general(onehot, hn, (((0,), (0,)), ((), ())),
                              preferred_element_type=jnp.float32)

    @pl.when(i == 0)
    def _():
        pool_ref[...] = jnp.zeros_like(pool_ref)

    pool_ref[...] += contrib


_mlp_call = pl.pallas_call(
    _mlp_body,
    grid=(NBLK,),
    in_specs=[
        pl.BlockSpec(memory_space=pltpu.SMEM),
        pl.BlockSpec((BLK, D), lambda i: (i, 0)),
        pl.BlockSpec((NC, BLK, D), lambda i: (0, i, 0)),
        pl.BlockSpec((D, D), lambda i: (0, 0)),
        pl.BlockSpec((1, D), lambda i: (0, 0)),
        pl.BlockSpec((D, D), lambda i: (0, 0)),
        pl.BlockSpec((1, D), lambda i: (0, 0)),
        pl.BlockSpec((BLK, 1), lambda i: (i, 0)),
    ],
    out_specs=[
        pl.BlockSpec((BLK, D), lambda i: (i, 0)),
        pl.BlockSpec((NG, D), lambda i: (0, 0)),
    ],
    out_shape=[
        jax.ShapeDtypeStruct((N_NODES, D), jnp.float32),
        jax.ShapeDtypeStruct((NG, D), jnp.float32),
    ],
    compiler_params=pltpu.CompilerParams(
        dimension_semantics=("arbitrary",)),
)


def _head_body(p0_ref, p1_ref, p2_ref, P1_ref, pb1_ref, P2_ref, pb2_ref, o_ref):
    a = (jnp.dot(p0_ref[...], P1_ref[0:D, :], preferred_element_type=jnp.float32)
         + jnp.dot(p1_ref[...], P1_ref[D:2 * D, :], preferred_element_type=jnp.float32)
         + jnp.dot(p2_ref[...], P1_ref[2 * D:3 * D, :], preferred_element_type=jnp.float32))
    a = jnp.maximum(a + pb1_ref[...], 0.0)
    o_ref[...] = (jnp.dot(a, P2_ref[...], preferred_element_type=jnp.float32)
                  + pb2_ref[...])


_head_call = pl.pallas_call(
    _head_body,
    out_shape=jax.ShapeDtypeStruct((NG, D), jnp.float32),
)


def kernel(x, edge_index, graph_ids, W1, b1, W2, b2, eps, P1, pb1, P2, pb2):
    src = edge_index[0]
    dst = edge_index[1]
    g2 = graph_ids.reshape(N_NODES, 1)
    h = x
    pooled = []
    for l in range(NL):
        parts = _agg_call(h, src, dst)
        h, pool_l = _mlp_call(eps[l].reshape(1, 1), h, parts,
                              W1[l], b1[l].reshape(1, D),
                              W2[l], b2[l].reshape(1, D), g2)
        pooled.append(pool_l)
    return _head_call(pooled[0], pooled[1], pooled[2],
                      P1, pb1.reshape(1, D), P2, pb2.reshape(1, D))


# speedup vs baseline: 6.3016x; 6.3016x over previous
import functools

import jax
import jax.numpy as jnp
from jax import lax
from jax.experimental import pallas as pl
from jax.experimental.pallas import tpu as pltpu
from jax.experimental.pallas import tpu_sc as plsc

N_NODES = 10000
N_EDGES = 320000
D = 128
NG = 128
NL = 3

NC = 2
NS = 16
NW = NC * NS

CHUNK = 128
NCHUNKS = N_EDGES // CHUNK
EDGE_ITERS = -(-NCHUNKS // NW)

ZROWS = 80
NZ = N_NODES // ZROWS
ZITERS = -(-NZ // NS)

ROWS_PER_TILE = 624


def _agg_body(h_hbm, src_hbm, dst_hbm, out_hbm, src_v, dst_v, rows_v, agg_sh, sem):
    c = lax.axis_index("c")
    s = lax.axis_index("s")
    wid = s * NC + c

    def zero_buf(i, carry):
        for j in range(D // 16):
            rows_v[i, pl.ds(j * 16, 16)] = jnp.zeros((16,), jnp.float32)
        return carry

    lax.fori_loop(0, CHUNK, zero_buf, 0)

    def zero_spmem(k, carry):
        cid = s + k * NS

        @pl.when(cid < NZ)
        def _():
            pltpu.sync_copy(rows_v.at[pl.ds(0, ZROWS)],
                            agg_sh.at[pl.ds(cid * ZROWS, ZROWS)])

        return carry

    lax.fori_loop(0, ZITERS, zero_spmem, 0)
    plsc.subcore_barrier()

    def edge_step(k, carry):
        cid = wid + k * NW

        @pl.when(cid < NCHUNKS)
        def _():
            base = cid * CHUNK
            pltpu.sync_copy(src_hbm.at[pl.ds(base, CHUNK)], src_v)
            pltpu.sync_copy(dst_hbm.at[pl.ds(base, CHUNK)], dst_v)
            pltpu.async_copy(h_hbm.at[src_v], rows_v, sem).wait()
            pltpu.sync_copy(rows_v, agg_sh.at[dst_v], add=True)

        return carry

    lax.fori_loop(0, EDGE_ITERS, edge_step, 0)
    plsc.subcore_barrier()

    r0 = s * ROWS_PER_TILE
    pltpu.sync_copy(agg_sh.at[pl.ds(r0, ROWS_PER_TILE)],
                    out_hbm.at[c, pl.ds(r0, ROWS_PER_TILE)])

    tail = NS * ROWS_PER_TILE

    @pl.when(s == NS - 1)
    def _():
        pltpu.sync_copy(agg_sh.at[pl.ds(tail, N_NODES - tail)],
                        out_hbm.at[c, pl.ds(tail, N_NODES - tail)])


_agg_call_cache = []


def _agg_call(h, src, dst):
    if not _agg_call_cache:
        _agg_call_cache.append(functools.partial(
            pl.kernel,
            out_type=jax.ShapeDtypeStruct((NC, N_NODES, D), jnp.float32),
            mesh=plsc.VectorSubcoreMesh(core_axis_name="c",
                                        subcore_axis_name="s"),
            scratch_types=[
                pltpu.VMEM((CHUNK,), jnp.int32),
                pltpu.VMEM((CHUNK,), jnp.int32),
                pltpu.VMEM((CHUNK, D), jnp.float32),
                pltpu.VMEM_SHARED((N_NODES, D), jnp.float32),
                pltpu.SemaphoreType.DMA,
            ],
        )(_agg_body))
    return _agg_call_cache[0](h, src, dst)


BLK = 1000
NBLK = N_NODES // BLK


def _mlp_body(eps_ref, h_ref, p_ref, w1_ref, b1_ref, w2_ref, b2_ref, g_ref,
              hn_ref, pool_ref):
    i = pl.program_id(0)
    h = h_ref[...]
    agg = p_ref[0, :, :] + p_ref[1, :, :]
    z = (1.0 + eps_ref[0, 0]) * h + agg
    z = jnp.maximum(
        jnp.dot(z, w1_ref[...], preferred_element_type=jnp.float32)
        + b1_ref[...], 0.0)
    hn = jnp.maximum(
        jnp.dot(z, w2_ref[...], preferred_element_type=jnp.float32)
        + b2_ref[...], 0.0)
    hn_ref[...] = hn

    cols = lax.broadcasted_iota(jnp.int32, (BLK, NG), 1)
    onehot = jnp.where(g_ref[...] == cols, 1.0, 0.0)
    contrib = lax.dot_general(onehot, hn, (((0,), (0,)), ((), ())),
                              preferred_element_type=jnp.float32)

    @pl.when(i == 0)
    def _():
        pool_ref[...] = jnp.zeros_like(pool_ref)

    pool_ref[...] += contrib


def _make_mlp_call(interpret=False):
    return pl.pallas_call(
        _mlp_body,
        grid=(NBLK,),
        in_specs=[
            pl.BlockSpec(memory_space=pltpu.SMEM),
            pl.BlockSpec((BLK, D), lambda i: (i, 0)),
            pl.BlockSpec((NC, BLK, D), lambda i: (0, i, 0)),
            pl.BlockSpec((D, D), lambda i: (0, 0)),
            pl.BlockSpec((1, D), lambda i: (0, 0)),
            pl.BlockSpec((D, D), lambda i: (0, 0)),
            pl.BlockSpec((1, D), lambda i: (0, 0)),
            pl.BlockSpec((BLK, 1), lambda i: (i, 0)),
        ],
        out_specs=[
            pl.BlockSpec((BLK, D), lambda i: (i, 0)),
            pl.BlockSpec((NG, D), lambda i: (0, 0)),
        ],
        out_shape=[
            jax.ShapeDtypeStruct((N_NODES, D), jnp.float32),
            jax.ShapeDtypeStruct((NG, D), jnp.float32),
        ],
        compiler_params=pltpu.CompilerParams(
            dimension_semantics=("arbitrary",)),
        interpret=interpret,
    )


_mlp_call = _make_mlp_call()


def _head_body(p0_ref, p1_ref, p2_ref, P1_ref, pb1_ref, P2_ref, pb2_ref, o_ref):
    a = (jnp.dot(p0_ref[...], P1_ref[0:D, :], preferred_element_type=jnp.float32)
         + jnp.dot(p1_ref[...], P1_ref[D:2 * D, :], preferred_element_type=jnp.float32)
         + jnp.dot(p2_ref[...], P1_ref[2 * D:3 * D, :], preferred_element_type=jnp.float32))
    a = jnp.maximum(a + pb1_ref[...], 0.0)
    o_ref[...] = (jnp.dot(a, P2_ref[...], preferred_element_type=jnp.float32)
                  + pb2_ref[...])


def _make_head_call(interpret=False):
    return pl.pallas_call(
        _head_body,
        out_shape=jax.ShapeDtypeStruct((NG, D), jnp.float32),
        interpret=interpret,
    )


_head_call = _make_head_call()


def kernel(x, edge_index, graph_ids, W1, b1, W2, b2, eps, P1, pb1, P2, pb2):
    src = edge_index[0]
    dst = edge_index[1]
    g2 = graph_ids.reshape(N_NODES, 1)
    h = x
    pooled = []
    for l in range(NL):
        parts = _agg_call(h, src, dst)
        h, pool_l = _mlp_call(eps[l].reshape(1, 1), h, parts,
                              W1[l], b1[l].reshape(1, D),
                              W2[l], b2[l].reshape(1, D), g2)
        pooled.append(pool_l)
    return _head_call(pooled[0], pooled[1], pooled[2],
                      P1, pb1.reshape(1, D), P2, pb2.reshape(1, D))
